# baseline (speedup 1.0000x reference)
import jax
import jax.numpy as jnp
from jax.experimental import pallas as pl
from jax.experimental.pallas import tpu as pltpu

_NBUF = 4


def _start_block_copy(adj_hbm, bufs, sems, blk, slot, bm):
    half = ((bm // 2) + 7) // 8 * 8
    base = blk * bm
    pltpu.make_async_copy(
        adj_hbm.at[pl.ds(base, half), :],
        bufs.at[slot, pl.ds(0, half), :],
        sems.at[slot, 0],
    ).start()
    pltpu.make_async_copy(
        adj_hbm.at[pl.ds(base + half, bm - half), :],
        bufs.at[slot, pl.ds(half, bm - half), :],
        sems.at[slot, 1],
    ).start()


def _wait_block_copy(adj_hbm, bufs, sems, blk, slot, bm):
    half = ((bm // 2) + 7) // 8 * 8
    base = blk * bm
    pltpu.make_async_copy(
        adj_hbm.at[pl.ds(base, half), :],
        bufs.at[slot, pl.ds(0, half), :],
        sems.at[slot, 0],
    ).wait()
    pltpu.make_async_copy(
        adj_hbm.at[pl.ds(base + half, bm - half), :],
        bufs.at[slot, pl.ds(half, bm - half), :],
        sems.at[slot, 1],
    ).wait()


def _gcn_kernel(x_ref, w_ref, b_ref, adj_hbm, out_ref, sup_ref, bufs, sems):
    m = pl.program_id(0)
    nblocks = pl.num_programs(0)
    bm = out_ref.shape[0]

    @pl.when(m == 0)
    def _():
        for i in range(_NBUF - 1):
            _start_block_copy(adj_hbm, bufs, sems, i, i, bm)
        sup_ref[...] = jnp.dot(
            x_ref[...], w_ref[...], preferred_element_type=jnp.float32
        )

    nxt = m + _NBUF - 1

    @pl.when(nxt < nblocks)
    def _():
        slot = jax.lax.rem(nxt, _NBUF)
        _start_block_copy(adj_hbm, bufs, sems, nxt, slot, bm)

    slot = jax.lax.rem(m, _NBUF)
    _wait_block_copy(adj_hbm, bufs, sems, m, slot, bm)
    out_ref[...] = (
        jnp.dot(bufs[slot], sup_ref[...], preferred_element_type=jnp.float32)
        + b_ref[...]
    )


def kernel(x, adj, weight, bias):
    n, d_in = x.shape
    d_out = weight.shape[1]
    bm = 200 if n % 200 == 0 else n
    b2 = bias.reshape(1, d_out)
    return pl.pallas_call(
        _gcn_kernel,
        grid=(n // bm,),
        in_specs=[
            pl.BlockSpec((n, d_in), lambda m: (0, 0)),
            pl.BlockSpec((d_in, d_out), lambda m: (0, 0)),
            pl.BlockSpec((1, d_out), lambda m: (0, 0)),
            pl.BlockSpec(memory_space=pl.ANY),
        ],
        out_specs=pl.BlockSpec((bm, d_out), lambda m: (m, 0)),
        out_shape=jax.ShapeDtypeStruct((n, d_out), jnp.float32),
        scratch_shapes=[
            pltpu.VMEM((n, d_out), jnp.float32),
            pltpu.VMEM((_NBUF, bm, n), jnp.float32),
            pltpu.SemaphoreType.DMA((_NBUF, 2)),
        ],
        compiler_params=pltpu.CompilerParams(
            dimension_semantics=("arbitrary",)
        ),
    )(x, weight, b2, adj)


# speedup vs baseline: 1.0078x; 1.0078x over previous
import jax
import jax.numpy as jnp
from jax.experimental import pallas as pl
from jax.experimental.pallas import tpu as pltpu


def _gcn_kernel(x_ref, w_ref, b_ref, adj_ref, out_ref, sup_ref):
    @pl.when(pl.program_id(0) == 0)
    def _():
        sup_ref[...] = jnp.dot(
            x_ref[...], w_ref[...], preferred_element_type=jnp.float32
        ).astype(jnp.bfloat16)

    out_ref[...] = (
        jnp.dot(
            adj_ref[...].astype(jnp.bfloat16),
            sup_ref[...],
            preferred_element_type=jnp.float32,
        )
        + b_ref[...]
    )


def kernel(x, adj, weight, bias):
    n, d_in = x.shape
    d_out = weight.shape[1]
    bm = 400 if n % 400 == 0 else n
    b2 = bias.reshape(1, d_out)
    return pl.pallas_call(
        _gcn_kernel,
        grid=(n // bm,),
        in_specs=[
            pl.BlockSpec((n, d_in), lambda m: (0, 0)),
            pl.BlockSpec((d_in, d_out), lambda m: (0, 0)),
            pl.BlockSpec((1, d_out), lambda m: (0, 0)),
            pl.BlockSpec((bm, n), lambda m: (m, 0)),
        ],
        out_specs=pl.BlockSpec((bm, d_out), lambda m: (m, 0)),
        out_shape=jax.ShapeDtypeStruct((n, d_out), jnp.float32),
        scratch_shapes=[pltpu.VMEM((n, d_out), jnp.bfloat16)],
        compiler_params=pltpu.CompilerParams(
            dimension_semantics=("arbitrary",)
        ),
    )(x, weight, b2, adj)
